# baseline (speedup 1.0000x reference)
import functools

import jax
import jax.numpy as jnp
from jax import lax
from jax.experimental import pallas as pl
from jax.experimental.pallas import tpu as pltpu
from jax.experimental.pallas import tpu_sc as plsc

_NC = 2
_NS = 16
_NW = _NC * _NS
_L = 16
_CHUNK = 128


def _sc_body(nch, d, ctable2, ptable2, xent, hidx, tidx, pidx, atom_out,
             x_v, hidx_v, tidx_v, pidx_v,
             hq_v, tq_v, pq_v, hp_v, tp_v, pp_v,
             hpair, tpair, ppair, prod, sem):
  rpw = nch * _CHUNK
  wid = lax.axis_index("s") * _NC + lax.axis_index("c")
  base = wid * rpw

  pltpu.sync_copy(xent, x_v)
  pltpu.sync_copy(hidx.at[wid], hidx_v)
  pltpu.sync_copy(tidx.at[wid], tidx_v)
  pltpu.sync_copy(pidx.at[wid], pidx_v)

  for j in range(nch):
    for i in range(_CHUNK // _L):
      s = pl.ds(i * _L, _L)
      hid = plsc.load_gather(x_v, [hidx_v[j, s]])
      tid = plsc.load_gather(x_v, [tidx_v[j, s]])
      pid = pidx_v[j, s]
      hq_v[j, s] = lax.shift_right_logical(hid, 1)
      tq_v[j, s] = lax.shift_right_logical(tid, 1)
      pq_v[j, s] = lax.shift_right_logical(pid, 1)
      hp_v[j, s] = lax.bitwise_and(hid, 1)
      tp_v[j, s] = lax.bitwise_and(tid, 1)
      pp_v[j, s] = lax.bitwise_and(pid, 1)

  for j in range(nch):
    cp_h = pltpu.async_copy(ctable2.at[hq_v.at[j]], hpair, sem)
    cp_t = pltpu.async_copy(ctable2.at[tq_v.at[j]], tpair, sem)
    cp_p = pltpu.async_copy(ptable2.at[pq_v.at[j]], ppair, sem)
    cp_h.wait()
    cp_t.wait()
    cp_p.wait()

    def group_body(g, carry):
      gs = pl.ds(g * _L, _L)
      hp16 = hp_v[j, gs]
      tp16 = tp_v[j, gs]
      pp16 = pp_v[j, gs]
      for lane in range(_L):
        r = g * _L + lane
        ho = hp16[lane] * d
        to = tp16[lane] * d
        po = pp16[lane] * d
        for c in range(d // _L):
          co = c * _L
          prod[r, pl.ds(co, _L)] = (
              hpair[r, pl.ds(ho + co, _L)]
              * tpair[r, pl.ds(to + co, _L)]
              * ppair[r, pl.ds(po + co, _L)])
      return carry

    lax.fori_loop(0, _CHUNK // _L, group_body, 0)

    pltpu.sync_copy(prod, atom_out.at[pl.ds(base + j * _CHUNK, _CHUNK)])


def _sc_gather_mul(ctable2, ptable2, xent, hidx, tidx, pidx, n, d):
  nch = (n // _NW) // _CHUNK
  mesh = plsc.VectorSubcoreMesh(
      core_axis_name="c", subcore_axis_name="s",
      num_cores=_NC, num_subcores=_NS)
  m = xent.shape[0]
  f = pl.kernel(
      functools.partial(_sc_body, nch, d),
      out_type=jax.ShapeDtypeStruct((n, d), jnp.float32),
      mesh=mesh,
      compiler_params=pltpu.CompilerParams(
          needs_layout_passes=False, use_tc_tiling_on_sc=True),
      scratch_types=[
          pltpu.VMEM((m,), jnp.int32),
          pltpu.VMEM((nch, _CHUNK), jnp.int32),
          pltpu.VMEM((nch, _CHUNK), jnp.int32),
          pltpu.VMEM((nch, _CHUNK), jnp.int32),
          pltpu.VMEM((nch, _CHUNK), jnp.int32),
          pltpu.VMEM((nch, _CHUNK), jnp.int32),
          pltpu.VMEM((nch, _CHUNK), jnp.int32),
          pltpu.VMEM((nch, _CHUNK), jnp.int32),
          pltpu.VMEM((nch, _CHUNK), jnp.int32),
          pltpu.VMEM((nch, _CHUNK), jnp.int32),
          pltpu.VMEM((_CHUNK, 2 * d), jnp.float32),
          pltpu.VMEM((_CHUNK, 2 * d), jnp.float32),
          pltpu.VMEM((_CHUNK, 2 * d), jnp.float32),
          pltpu.VMEM((_CHUNK, d), jnp.float32),
          pltpu.SemaphoreType.DMA,
      ],
  )
  return f(ctable2, ptable2, xent, hidx, tidx, pidx)


def _mm_body(atom_ref, w_ref, b_ref, o_ref):
  o_ref[...] = (
      jnp.dot(atom_ref[...], w_ref[...], preferred_element_type=jnp.float32)
      + b_ref[...])


def _out_proj(atom, w, b):
  n, d = atom.shape
  bm = 2048
  return pl.pallas_call(
      _mm_body,
      grid=(n // bm,),
      in_specs=[
          pl.BlockSpec((bm, d), lambda i: (i, 0)),
          pl.BlockSpec((d, d), lambda i: (0, 0)),
          pl.BlockSpec((1, d), lambda i: (0, 0)),
      ],
      out_specs=pl.BlockSpec((bm, d), lambda i: (i, 0)),
      out_shape=jax.ShapeDtypeStruct((n, d), jnp.float32),
  )(atom, w, b.reshape(1, d))


def kernel(constant_table, predicate_table, W_out, b_out, x_entity,
           pred_ids, const_idx):
  n = pred_ids.shape[0]
  d = constant_table.shape[1]
  nch = (n // _NW) // _CHUNK
  ctable2 = constant_table.reshape(constant_table.shape[0] // 2, 2 * d)
  ptable2 = predicate_table.reshape(predicate_table.shape[0] // 2, 2 * d)
  xent = x_entity.astype(jnp.int32)
  hidx = const_idx[:, 0].astype(jnp.int32).reshape(_NW, nch, _CHUNK)
  tidx = const_idx[:, 1].astype(jnp.int32).reshape(_NW, nch, _CHUNK)
  pidx = pred_ids.astype(jnp.int32).reshape(_NW, nch, _CHUNK)
  atom = _sc_gather_mul(ctable2, ptable2, xent, hidx, tidx, pidx, n, d)
  return _out_proj(atom, W_out, b_out)
